# baseline (speedup 1.0000x reference)
import jax
import jax.numpy as jnp
from jax.experimental import pallas as pl

_ROWS, _VOCAB = 128, 100000
_RB = 8
_TINY = 1.1754943508222875e-38


def _threefry_bits(idx):
    ks0 = jnp.uint32(0)
    ks1 = jnp.uint32(42)
    ks2 = jnp.uint32(466689008)
    x1 = jnp.zeros_like(idx)
    x2 = idx + ks1
    rots = ((13, 15, 26, 6), (17, 29, 16, 24))
    inject = ((ks1, ks2, 1), (ks2, ks0, 2), (ks0, ks1, 3),
              (ks1, ks2, 4), (ks2, ks0, 5))
    for g in range(5):
        for r in rots[g % 2]:
            x1 = x1 + x2
            x2 = (x2 << jnp.uint32(r)) | (x2 >> jnp.uint32(32 - r))
            x2 = x2 ^ x1
        a, b, i = inject[g]
        x1 = x1 + a
        x2 = x2 + b + jnp.uint32(i)
    return x1 ^ x2


def _body(scores_ref, out_ref):
    pid = pl.program_id(0)
    row0 = pid * _RB
    r = jax.lax.broadcasted_iota(jnp.int32, (_RB, _VOCAB), 0)
    c = jax.lax.broadcasted_iota(jnp.int32, (_RB, _VOCAB), 1)
    idx = ((row0 + r) * _VOCAB + c).astype(jnp.uint32)

    bits = _threefry_bits(idx)
    tiny = jnp.float32(_TINY)
    fb = (bits >> jnp.uint32(9)) | jnp.uint32(0x3F800000)
    u = jax.lax.bitcast_convert_type(fb, jnp.float32) - jnp.float32(1.0)
    u = u * (jnp.float32(1.0) - tiny) + tiny
    u = jnp.maximum(tiny, u)
    g = -jnp.log(-jnp.log(u))

    s = scores_ref[...] + g
    m1 = jnp.max(s, axis=1, keepdims=True)
    e1 = jnp.exp(s - m1)
    z1 = jnp.sum(e1, axis=1, keepdims=True)
    p1 = e1 / z1
    s2 = s + jnp.log(jnp.maximum(jnp.float32(1.0) - p1, tiny))
    m2 = jnp.max(s2, axis=1, keepdims=True)
    e2 = jnp.exp(s2 - m2)
    z2 = jnp.sum(e2, axis=1, keepdims=True)
    khot = p1 + e2 / z2

    v1 = jnp.max(khot, axis=1, keepdims=True)
    i1 = jnp.min(jnp.where(khot == v1, c, _VOCAB), axis=1, keepdims=True)
    khot_m = jnp.where(c == i1, -jnp.inf, khot)
    v2 = jnp.max(khot_m, axis=1, keepdims=True)
    i2 = jnp.min(jnp.where(khot_m == v2, c, _VOCAB), axis=1, keepdims=True)
    sel = (c == i1) | (c == i2)
    out_ref[...] = jnp.where(sel, (jnp.float32(1.0) - khot) + khot,
                             jnp.float32(0.0))


def _sampler(scores, interpret=False):
    return pl.pallas_call(
        _body,
        grid=(_ROWS // _RB,),
        in_specs=[pl.BlockSpec((_RB, _VOCAB), lambda i: (i, 0))],
        out_specs=pl.BlockSpec((_RB, _VOCAB), lambda i: (i, 0)),
        out_shape=jax.ShapeDtypeStruct((_ROWS, _VOCAB), jnp.float32),
        interpret=interpret,
    )(scores)


def kernel(scores):
    return _sampler(scores.astype(jnp.float32))


# speedup vs baseline: 1.1278x; 1.1278x over previous
import jax
import jax.numpy as jnp
from jax.experimental import pallas as pl

_ROWS, _VOCAB = 128, 100000
_RB = 8
_TINY = 1.1754943508222875e-38


def _threefry_bits(idx):
    ks0 = jnp.uint32(0)
    ks1 = jnp.uint32(42)
    ks2 = jnp.uint32(466689008)
    x1 = jnp.zeros_like(idx)
    x2 = idx + ks1
    rots = ((13, 15, 26, 6), (17, 29, 16, 24))
    inject = ((ks1, ks2, 1), (ks2, ks0, 2), (ks0, ks1, 3),
              (ks1, ks2, 4), (ks2, ks0, 5))
    for g in range(5):
        for r in rots[g % 2]:
            x1 = x1 + x2
            x2 = (x2 << jnp.uint32(r)) | (x2 >> jnp.uint32(32 - r))
            x2 = x2 ^ x1
        a, b, i = inject[g]
        x1 = x1 + a
        x2 = x2 + b + jnp.uint32(i)
    return x1 ^ x2


def _body(scores_ref, out_ref):
    pid = pl.program_id(0)
    row0 = pid * _RB
    r = jax.lax.broadcasted_iota(jnp.int32, (_RB, _VOCAB), 0)
    c = jax.lax.broadcasted_iota(jnp.int32, (_RB, _VOCAB), 1)
    idx = ((row0 + r) * _VOCAB + c).astype(jnp.uint32)

    bits = _threefry_bits(idx)
    tiny = jnp.float32(_TINY)
    fb = (bits >> jnp.uint32(9)) | jnp.uint32(0x3F800000)
    u = jax.lax.bitcast_convert_type(fb, jnp.float32) - jnp.float32(1.0)
    u = u * (jnp.float32(1.0) - tiny) + tiny
    u = jnp.maximum(tiny, u)
    g = -jnp.log(-jnp.log(u))

    s = scores_ref[...] + g
    v1 = jnp.max(s, axis=1, keepdims=True)
    i1 = jnp.min(jnp.where(s == v1, c, _VOCAB), axis=1, keepdims=True)
    s_m = jnp.where(c == i1, -jnp.inf, s)
    v2 = jnp.max(s_m, axis=1, keepdims=True)
    i2 = jnp.min(jnp.where(s_m == v2, c, _VOCAB), axis=1, keepdims=True)
    sel = (c == i1) | (c == i2)
    out_ref[...] = jnp.where(sel, jnp.float32(1.0), jnp.float32(0.0))


def _sampler(scores, interpret=False):
    return pl.pallas_call(
        _body,
        grid=(_ROWS // _RB,),
        in_specs=[pl.BlockSpec((_RB, _VOCAB), lambda i: (i, 0))],
        out_specs=pl.BlockSpec((_RB, _VOCAB), lambda i: (i, 0)),
        out_shape=jax.ShapeDtypeStruct((_ROWS, _VOCAB), jnp.float32),
        interpret=interpret,
    )(scores)


def kernel(scores):
    return _sampler(scores.astype(jnp.float32))
